# baseline (speedup 1.0000x reference)
import jax
import jax.numpy as jnp
from jax.experimental import pallas as pl
from jax.experimental.pallas import tpu as pltpu

_CB = 8192
_D = 32
_BR = 256
_NBLK = 8192 // _BR
_N_TOTAL = 8192
_HALF = _CB // 2
_COMMIT = 0.25


def _norm32(v):
    t = ((v[..., 0:8] + v[..., 8:16]) + v[..., 16:24]) + v[..., 24:32]
    t = t[..., 0:4] + t[..., 4:8]
    t = t[..., 0:2] + t[..., 2:4]
    return t[..., 0:1] + t[..., 1:2]


def _vq_block(x_ref, emb_ref, enc_ref, qst_ref, loss_ref, perp_ref,
              counts_ref, lsum_ref):
    i = pl.program_id(0)
    x = x_ref[...]
    emb = emb_ref[...]

    xn = _norm32(x * x)
    en = _norm32(emb * emb).reshape(1, _CB)
    m = jax.lax.dot_general(x.astype(jnp.bfloat16), emb.astype(jnp.bfloat16),
                            (((1,), (1,)), ((), ())),
                            preferred_element_type=jnp.float32)
    d = (xn + en) - 2.0 * m

    d1 = d[:, :_HALF]
    d2 = d[:, _HALF:]
    lane = jax.lax.broadcasted_iota(jnp.int32, (_BR, _HALF), 1)
    m1 = jnp.min(d1, axis=1, keepdims=True)
    i1 = jnp.min(jnp.where(d1 == m1, lane, _CB), axis=1, keepdims=True)
    m2 = jnp.min(d2, axis=1, keepdims=True)
    i2 = jnp.min(jnp.where(d2 == m2, lane, _CB), axis=1, keepdims=True) + _HALF
    m1b = m1.astype(jnp.bfloat16).astype(jnp.float32)
    idx = jnp.where(m2 < m1b, i2, i1)

    lane_full = jax.lax.broadcasted_iota(jnp.int32, (_BR, _CB), 1)
    onehot = (lane_full == idx).astype(jnp.float32)
    enc_ref[...] = onehot

    q = jax.lax.dot_general(onehot.astype(jnp.bfloat16),
                            emb.astype(jnp.bfloat16),
                            (((1,), (0,)), ((), ())),
                            preferred_element_type=jnp.float32)
    qst_ref[...] = x + (q - x)

    @pl.when(i == 0)
    def _init():
        lsum_ref[0, 0] = 0.0
        counts_ref[...] = jnp.zeros_like(counts_ref)

    lsum_ref[0, 0] += jnp.sum((q - x) * (q - x))
    counts_ref[...] += jnp.sum(onehot, axis=0, keepdims=True)

    @pl.when(i == _NBLK - 1)
    def _finish():
        v = lsum_ref[0, 0] / (_N_TOTAL * _D)
        loss_ref[...] = jnp.full((1, 1), v + _COMMIT * v, jnp.float32)
        avg = counts_ref[...] / _N_TOTAL
        ent = -jnp.sum(avg * jnp.log(avg + 1e-10))
        perp_ref[...] = jnp.full((1, 1), jnp.exp(ent), jnp.float32)


def kernel(inputs, embedding_weight):
    input_shape = inputs.shape
    flat = inputs.reshape(-1, _D)

    enc, qst, loss, perp = pl.pallas_call(
        _vq_block,
        grid=(_NBLK,),
        in_specs=[
            pl.BlockSpec((_BR, _D), lambda i: (i, 0)),
            pl.BlockSpec((_CB, _D), lambda i: (0, 0)),
        ],
        out_specs=[
            pl.BlockSpec((_BR, _CB), lambda i: (i, 0)),
            pl.BlockSpec((_BR, _D), lambda i: (i, 0)),
            pl.BlockSpec((1, 1), lambda i: (0, 0)),
            pl.BlockSpec((1, 1), lambda i: (0, 0)),
        ],
        out_shape=[
            jax.ShapeDtypeStruct((_N_TOTAL, _CB), jnp.float32),
            jax.ShapeDtypeStruct((_N_TOTAL, _D), jnp.float32),
            jax.ShapeDtypeStruct((1, 1), jnp.float32),
            jax.ShapeDtypeStruct((1, 1), jnp.float32),
        ],
        scratch_shapes=[
            pltpu.VMEM((1, _CB), jnp.float32),
            pltpu.SMEM((1, 1), jnp.float32),
        ],
        compiler_params=pltpu.CompilerParams(
            dimension_semantics=("arbitrary",)),
    )(flat, embedding_weight)

    return (loss.reshape(()), qst.reshape(input_shape), perp.reshape(()), enc)


# speedup vs baseline: 2.4064x; 2.4064x over previous
import jax
import jax.numpy as jnp
from jax.experimental import pallas as pl
from jax.experimental.pallas import tpu as pltpu

_CB = 8192
_D = 32
_BR = 256
_NBLK = 8192 // _BR
_N_TOTAL = 8192
_HALF = _CB // 2
_COMMIT = 0.25


def _norm32(v, axis):
    def s(a, b):
        return jax.lax.slice_in_dim(v, a, b, axis=axis)
    t = ((s(0, 8) + s(8, 16)) + s(16, 24)) + s(24, 32)
    def h(u, a, b):
        return jax.lax.slice_in_dim(u, a, b, axis=axis)
    t = h(t, 0, 4) + h(t, 4, 8)
    t = h(t, 0, 2) + h(t, 2, 4)
    return h(t, 0, 1) + h(t, 1, 2)


def _vq_block(x_ref, emb_ref, embt_ref, enc_ref, qst_ref, loss_ref, perp_ref,
              en_ref, counts_ref, lsum_ref):
    i = pl.program_id(0)
    x = x_ref[...]

    @pl.when(i == 0)
    def _init():
        et = embt_ref[...]
        en_ref[...] = _norm32(et * et, axis=0)
        lsum_ref[0, 0] = 0.0
        counts_ref[...] = jnp.zeros_like(counts_ref)

    xn = _norm32(x * x, axis=1)
    m2 = jax.lax.dot_general((x + x).astype(jnp.bfloat16),
                             emb_ref[...].astype(jnp.bfloat16),
                             (((1,), (1,)), ((), ())),
                             preferred_element_type=jnp.float32)
    d = (xn + en_ref[...]) - m2

    d1 = d[:, :_HALF]
    d2 = d[:, _HALF:]
    lane = jax.lax.broadcasted_iota(jnp.int32, (_BR, _HALF), 1)
    m1 = jnp.min(d1, axis=1, keepdims=True)
    i1 = jnp.min(jnp.where(d1 == m1, lane, _CB), axis=1, keepdims=True)
    m2w = jnp.min(d2, axis=1, keepdims=True)
    i2 = jnp.min(jnp.where(d2 == m2w, lane, _CB), axis=1, keepdims=True) + _HALF
    m1b = m1.astype(jnp.bfloat16).astype(jnp.float32)
    idx = jnp.where(m2w < m1b, i2, i1)

    lane_full = jax.lax.broadcasted_iota(jnp.int32, (_BR, _CB), 1)
    mask = lane_full == idx
    enc_ref[...] = mask.astype(jnp.float32)
    oh_bf = mask.astype(jnp.bfloat16)

    q = jax.lax.dot_general(oh_bf, emb_ref[...].astype(jnp.bfloat16),
                            (((1,), (0,)), ((), ())),
                            preferred_element_type=jnp.float32)
    qst_ref[...] = x + (q - x)

    ones_row = jnp.ones((1, _BR), jnp.bfloat16)
    counts_ref[...] += jax.lax.dot_general(
        ones_row, oh_bf, (((1,), (0,)), ((), ())),
        preferred_element_type=jnp.float32)
    lsum_ref[0, 0] += jnp.sum((q - x) * (q - x))

    @pl.when(i == _NBLK - 1)
    def _finish():
        v = lsum_ref[0, 0] / (_N_TOTAL * _D)
        loss_ref[...] = jnp.full((1, 1), v + _COMMIT * v, jnp.float32)
        avg = counts_ref[...] / _N_TOTAL
        ent = -jnp.sum(avg * jnp.log(avg + 1e-10))
        perp_ref[...] = jnp.full((1, 1), jnp.exp(ent), jnp.float32)


def kernel(inputs, embedding_weight):
    input_shape = inputs.shape
    flat = inputs.reshape(-1, _D)
    emb_t = embedding_weight.T

    enc, qst, loss, perp = pl.pallas_call(
        _vq_block,
        grid=(_NBLK,),
        in_specs=[
            pl.BlockSpec((_BR, _D), lambda i: (i, 0)),
            pl.BlockSpec((_CB, _D), lambda i: (0, 0)),
            pl.BlockSpec((_D, _CB), lambda i: (0, 0)),
        ],
        out_specs=[
            pl.BlockSpec((_BR, _CB), lambda i: (i, 0)),
            pl.BlockSpec((_BR, _D), lambda i: (i, 0)),
            pl.BlockSpec((1, 1), lambda i: (0, 0)),
            pl.BlockSpec((1, 1), lambda i: (0, 0)),
        ],
        out_shape=[
            jax.ShapeDtypeStruct((_N_TOTAL, _CB), jnp.float32),
            jax.ShapeDtypeStruct((_N_TOTAL, _D), jnp.float32),
            jax.ShapeDtypeStruct((1, 1), jnp.float32),
            jax.ShapeDtypeStruct((1, 1), jnp.float32),
        ],
        scratch_shapes=[
            pltpu.VMEM((1, _CB), jnp.float32),
            pltpu.VMEM((1, _CB), jnp.float32),
            pltpu.SMEM((1, 1), jnp.float32),
        ],
        compiler_params=pltpu.CompilerParams(
            dimension_semantics=("arbitrary",)),
    )(flat, embedding_weight, emb_t)

    return (loss.reshape(()), qst.reshape(input_shape), perp.reshape(()), enc)


# speedup vs baseline: 2.4349x; 1.0118x over previous
import jax
import jax.numpy as jnp
from jax.experimental import pallas as pl
from jax.experimental.pallas import tpu as pltpu

_CB = 8192
_D = 32
_BR = 256
_NBLK = 8192 // _BR
_N_TOTAL = 8192
_HALF = _CB // 2
_COMMIT = 0.25


def _norm32(v, axis):
    def s(a, b):
        return jax.lax.slice_in_dim(v, a, b, axis=axis)
    t = ((s(0, 8) + s(8, 16)) + s(16, 24)) + s(24, 32)
    def h(u, a, b):
        return jax.lax.slice_in_dim(u, a, b, axis=axis)
    t = h(t, 0, 4) + h(t, 4, 8)
    t = h(t, 0, 2) + h(t, 2, 4)
    return h(t, 0, 1) + h(t, 1, 2)


def _vq_block(x_ref, emb_ref, embt_ref, enc_ref, qst_ref, loss_ref, perp_ref,
              en_ref, counts_ref, lacc_ref):
    i = pl.program_id(0)
    x = x_ref[...]

    @pl.when(i == 0)
    def _init():
        et = embt_ref[...]
        en_ref[...] = _norm32(et * et, axis=0)
        lacc_ref[...] = jnp.zeros_like(lacc_ref)
        counts_ref[...] = jnp.zeros_like(counts_ref)

    xn = _norm32(x * x, axis=1)
    m2 = jax.lax.dot_general((x + x).astype(jnp.bfloat16),
                             emb_ref[...].astype(jnp.bfloat16),
                             (((1,), (1,)), ((), ())),
                             preferred_element_type=jnp.float32)
    d = (xn + en_ref[...]) - m2

    d1 = d[:, :_HALF]
    d2 = d[:, _HALF:]
    lane = jax.lax.broadcasted_iota(jnp.int32, (_BR, _HALF), 1)
    m1 = jnp.min(d1, axis=1, keepdims=True)
    i1 = jnp.min(jnp.where(d1 == m1, lane, _CB), axis=1, keepdims=True)
    m2w = jnp.min(d2, axis=1, keepdims=True)
    i2 = jnp.min(jnp.where(d2 == m2w, lane, _CB), axis=1, keepdims=True) + _HALF
    m1b = m1.astype(jnp.bfloat16).astype(jnp.float32)
    idx = jnp.where(m2w < m1b, i2, i1)

    lane_full = jax.lax.broadcasted_iota(jnp.int32, (_BR, _CB), 1)
    mask = lane_full == idx
    enc_ref[...] = mask.astype(jnp.float32)
    oh_bf = mask.astype(jnp.bfloat16)

    q = jax.lax.dot_general(oh_bf, emb_ref[...].astype(jnp.bfloat16),
                            (((1,), (0,)), ((), ())),
                            preferred_element_type=jnp.float32)
    delta = q - x
    qst_ref[...] = x + delta

    ones_row = jnp.ones((1, _BR), jnp.bfloat16)
    counts_ref[...] += jax.lax.dot_general(
        ones_row, oh_bf, (((1,), (0,)), ((), ())),
        preferred_element_type=jnp.float32)
    lacc_ref[...] += delta * delta

    @pl.when(i == _NBLK - 1)
    def _finish():
        v = jnp.sum(lacc_ref[...]) / (_N_TOTAL * _D)
        loss_ref[...] = jnp.full((1, 1), v + _COMMIT * v, jnp.float32)
        avg = counts_ref[...] / _N_TOTAL
        ent = -jnp.sum(avg * jnp.log(avg + 1e-10))
        perp_ref[...] = jnp.full((1, 1), jnp.exp(ent), jnp.float32)


def kernel(inputs, embedding_weight):
    input_shape = inputs.shape
    flat = inputs.reshape(-1, _D)
    emb_t = embedding_weight.T

    enc, qst, loss, perp = pl.pallas_call(
        _vq_block,
        grid=(_NBLK,),
        in_specs=[
            pl.BlockSpec((_BR, _D), lambda i: (i, 0)),
            pl.BlockSpec((_CB, _D), lambda i: (0, 0)),
            pl.BlockSpec((_D, _CB), lambda i: (0, 0)),
        ],
        out_specs=[
            pl.BlockSpec((_BR, _CB), lambda i: (i, 0)),
            pl.BlockSpec((_BR, _D), lambda i: (i, 0)),
            pl.BlockSpec((1, 1), lambda i: (0, 0)),
            pl.BlockSpec((1, 1), lambda i: (0, 0)),
        ],
        out_shape=[
            jax.ShapeDtypeStruct((_N_TOTAL, _CB), jnp.float32),
            jax.ShapeDtypeStruct((_N_TOTAL, _D), jnp.float32),
            jax.ShapeDtypeStruct((1, 1), jnp.float32),
            jax.ShapeDtypeStruct((1, 1), jnp.float32),
        ],
        scratch_shapes=[
            pltpu.VMEM((1, _CB), jnp.float32),
            pltpu.VMEM((1, _CB), jnp.float32),
            pltpu.VMEM((_BR, _D), jnp.float32),
        ],
        compiler_params=pltpu.CompilerParams(
            dimension_semantics=("arbitrary",)),
    )(flat, embedding_weight, emb_t)

    return (loss.reshape(()), qst.reshape(input_shape), perp.reshape(()), enc)


# speedup vs baseline: 2.4445x; 1.0039x over previous
import jax
import jax.numpy as jnp
from jax.experimental import pallas as pl
from jax.experimental.pallas import tpu as pltpu

_CB = 8192
_D = 32
_BR = 256
_NBLK = 8192 // _BR
_N_TOTAL = 8192
_HALF = _CB // 2
_COMMIT = 0.25


def _norm32(v, axis):
    def s(a, b):
        return jax.lax.slice_in_dim(v, a, b, axis=axis)
    t = ((s(0, 8) + s(8, 16)) + s(16, 24)) + s(24, 32)
    def h(u, a, b):
        return jax.lax.slice_in_dim(u, a, b, axis=axis)
    t = h(t, 0, 4) + h(t, 4, 8)
    t = h(t, 0, 2) + h(t, 2, 4)
    return h(t, 0, 1) + h(t, 1, 2)


def _vq_block(x_ref, emb_ref, embt_ref, enc_ref, qst_ref, loss_ref, perp_ref,
              en_ref, embbf_ref, counts_ref, lacc_ref):
    i = pl.program_id(0)
    x = x_ref[...]

    @pl.when(i == 0)
    def _init():
        et = embt_ref[...]
        en_ref[...] = _norm32(et * et, axis=0)
        embbf_ref[...] = emb_ref[...].astype(jnp.bfloat16)
        lacc_ref[...] = jnp.zeros_like(lacc_ref)
        counts_ref[...] = jnp.zeros_like(counts_ref)

    xn = _norm32(x * x, axis=1)
    m2 = jax.lax.dot_general((x + x).astype(jnp.bfloat16), embbf_ref[...],
                             (((1,), (1,)), ((), ())),
                             preferred_element_type=jnp.float32)
    d = (xn + en_ref[...]) - m2

    d1 = d[:, :_HALF]
    d2 = d[:, _HALF:]
    lane = jax.lax.broadcasted_iota(jnp.int32, (_BR, _HALF), 1)
    m1 = jnp.min(d1, axis=1, keepdims=True)
    i1 = jnp.min(jnp.where(d1 == m1, lane, _CB), axis=1, keepdims=True)
    m2w = jnp.min(d2, axis=1, keepdims=True)
    i2 = jnp.min(jnp.where(d2 == m2w, lane, _CB), axis=1, keepdims=True) + _HALF
    m1b = m1.astype(jnp.bfloat16).astype(jnp.float32)
    idx = jnp.where(m2w < m1b, i2, i1)

    lane_full = jax.lax.broadcasted_iota(jnp.int32, (_BR, _CB), 1)
    mask = lane_full == idx
    enc_ref[...] = mask.astype(jnp.float32)
    oh_bf = mask.astype(jnp.bfloat16)

    q = jax.lax.dot_general(oh_bf, embbf_ref[...],
                            (((1,), (0,)), ((), ())),
                            preferred_element_type=jnp.float32)
    delta = q - x
    qst_ref[...] = x + delta

    ones_row = jnp.ones((1, _BR), jnp.bfloat16)
    counts_ref[...] += jax.lax.dot_general(
        ones_row, oh_bf, (((1,), (0,)), ((), ())),
        preferred_element_type=jnp.float32)
    lacc_ref[...] += delta * delta

    @pl.when(i == _NBLK - 1)
    def _finish():
        v = jnp.sum(lacc_ref[...]) / (_N_TOTAL * _D)
        loss_ref[...] = jnp.full((1, 1), v + _COMMIT * v, jnp.float32)
        avg = counts_ref[...] / _N_TOTAL
        ent = -jnp.sum(avg * jnp.log(avg + 1e-10))
        perp_ref[...] = jnp.full((1, 1), jnp.exp(ent), jnp.float32)


def kernel(inputs, embedding_weight):
    input_shape = inputs.shape
    flat = inputs.reshape(-1, _D)
    emb_t = embedding_weight.T

    enc, qst, loss, perp = pl.pallas_call(
        _vq_block,
        grid=(_NBLK,),
        in_specs=[
            pl.BlockSpec((_BR, _D), lambda i: (i, 0)),
            pl.BlockSpec((_CB, _D), lambda i: (0, 0)),
            pl.BlockSpec((_D, _CB), lambda i: (0, 0)),
        ],
        out_specs=[
            pl.BlockSpec((_BR, _CB), lambda i: (i, 0)),
            pl.BlockSpec((_BR, _D), lambda i: (i, 0)),
            pl.BlockSpec((1, 1), lambda i: (0, 0)),
            pl.BlockSpec((1, 1), lambda i: (0, 0)),
        ],
        out_shape=[
            jax.ShapeDtypeStruct((_N_TOTAL, _CB), jnp.float32),
            jax.ShapeDtypeStruct((_N_TOTAL, _D), jnp.float32),
            jax.ShapeDtypeStruct((1, 1), jnp.float32),
            jax.ShapeDtypeStruct((1, 1), jnp.float32),
        ],
        scratch_shapes=[
            pltpu.VMEM((1, _CB), jnp.float32),
            pltpu.VMEM((_CB, _D), jnp.bfloat16),
            pltpu.VMEM((1, _CB), jnp.float32),
            pltpu.VMEM((_BR, _D), jnp.float32),
        ],
        compiler_params=pltpu.CompilerParams(
            dimension_semantics=("arbitrary",)),
    )(flat, embedding_weight, emb_t)

    return (loss.reshape(()), qst.reshape(input_shape), perp.reshape(()), enc)


# speedup vs baseline: 2.4831x; 1.0158x over previous
import jax
import jax.numpy as jnp
from jax.experimental import pallas as pl
from jax.experimental.pallas import tpu as pltpu

_CB = 8192
_D = 32
_BR = 256
_NBLK = 8192 // _BR
_N_TOTAL = 8192
_HALF = _CB // 2
_COMMIT = 0.25


def _norm32(v, axis):
    def s(a, b):
        return jax.lax.slice_in_dim(v, a, b, axis=axis)
    t = ((s(0, 8) + s(8, 16)) + s(16, 24)) + s(24, 32)
    def h(u, a, b):
        return jax.lax.slice_in_dim(u, a, b, axis=axis)
    t = h(t, 0, 4) + h(t, 4, 8)
    t = h(t, 0, 2) + h(t, 2, 4)
    return h(t, 0, 1) + h(t, 1, 2)


def _vq_block(x_ref, emb_ref, embt_ref, enc_ref, qst_ref, loss_ref, perp_ref,
              en_ref, embbf_ref, counts_ref, lacc_ref):
    i = pl.program_id(0)
    x = x_ref[...]

    @pl.when(i == 0)
    def _init():
        et = embt_ref[...]
        en_ref[...] = _norm32(et * et, axis=0)
        embbf_ref[...] = emb_ref[...].astype(jnp.bfloat16)
        lacc_ref[...] = jnp.zeros_like(lacc_ref)
        counts_ref[...] = jnp.zeros_like(counts_ref)

    xn = _norm32(x * x, axis=1)
    m2 = jax.lax.dot_general((x + x).astype(jnp.bfloat16), embbf_ref[...],
                             (((1,), (1,)), ((), ())),
                             preferred_element_type=jnp.float32)
    d = (xn + en_ref[...]) - m2

    d1 = d[:, :_HALF]
    d2 = d[:, _HALF:]
    lane = jax.lax.broadcasted_iota(
        jnp.int32, (_BR, _HALF), 1).astype(jnp.float32)
    big = jnp.float32(_CB)
    m1 = jnp.min(d1, axis=1, keepdims=True)
    i1 = jnp.min(jnp.where(d1 == m1, lane, big), axis=1, keepdims=True)
    m2w = jnp.min(d2, axis=1, keepdims=True)
    i2 = jnp.min(jnp.where(d2 == m2w, lane, big), axis=1, keepdims=True) + _HALF
    m1b = m1.astype(jnp.bfloat16).astype(jnp.float32)
    idx = jnp.where(m2w < m1b, i2, i1).astype(jnp.int32)

    lane_full = jax.lax.broadcasted_iota(jnp.int32, (_BR, _CB), 1)
    mask = lane_full == idx
    enc_ref[...] = mask.astype(jnp.float32)
    oh_bf = mask.astype(jnp.bfloat16)

    q = jax.lax.dot_general(oh_bf, embbf_ref[...],
                            (((1,), (0,)), ((), ())),
                            preferred_element_type=jnp.float32)
    delta = q - x
    qst_ref[...] = x + delta

    ones_row = jnp.ones((1, _BR), jnp.bfloat16)
    counts_ref[...] += jax.lax.dot_general(
        ones_row, oh_bf, (((1,), (0,)), ((), ())),
        preferred_element_type=jnp.float32)
    lacc_ref[...] += delta * delta

    @pl.when(i == _NBLK - 1)
    def _finish():
        v = jnp.sum(lacc_ref[...]) / (_N_TOTAL * _D)
        loss_ref[...] = jnp.full((1, 1), v + _COMMIT * v, jnp.float32)
        avg = counts_ref[...] / _N_TOTAL
        ent = -jnp.sum(avg * jnp.log(avg + 1e-10))
        perp_ref[...] = jnp.full((1, 1), jnp.exp(ent), jnp.float32)


def kernel(inputs, embedding_weight):
    input_shape = inputs.shape
    flat = inputs.reshape(-1, _D)
    emb_t = embedding_weight.T

    enc, qst, loss, perp = pl.pallas_call(
        _vq_block,
        grid=(_NBLK,),
        in_specs=[
            pl.BlockSpec((_BR, _D), lambda i: (i, 0)),
            pl.BlockSpec((_CB, _D), lambda i: (0, 0)),
            pl.BlockSpec((_D, _CB), lambda i: (0, 0)),
        ],
        out_specs=[
            pl.BlockSpec((_BR, _CB), lambda i: (i, 0)),
            pl.BlockSpec((_BR, _D), lambda i: (i, 0)),
            pl.BlockSpec((1, 1), lambda i: (0, 0)),
            pl.BlockSpec((1, 1), lambda i: (0, 0)),
        ],
        out_shape=[
            jax.ShapeDtypeStruct((_N_TOTAL, _CB), jnp.float32),
            jax.ShapeDtypeStruct((_N_TOTAL, _D), jnp.float32),
            jax.ShapeDtypeStruct((1, 1), jnp.float32),
            jax.ShapeDtypeStruct((1, 1), jnp.float32),
        ],
        scratch_shapes=[
            pltpu.VMEM((1, _CB), jnp.float32),
            pltpu.VMEM((_CB, _D), jnp.bfloat16),
            pltpu.VMEM((1, _CB), jnp.float32),
            pltpu.VMEM((_BR, _D), jnp.float32),
        ],
        compiler_params=pltpu.CompilerParams(
            dimension_semantics=("arbitrary",)),
    )(flat, embedding_weight, emb_t)

    return (loss.reshape(()), qst.reshape(input_shape), perp.reshape(()), enc)
